# baseline (speedup 1.0000x reference)
import numpy as np
import jax
import jax.numpy as jnp
from jax import lax
from jax.experimental import pallas as pl
from jax.experimental.pallas import tpu as pltpu

_N = 16
_S = 4096
_D = 256
_E = _N * (_N - 1) // 2
_EP = 128
_K = max(int(max(0.5 * 0.999, 0.4) * _E), 1)
_RF = 0.8
_CHUNK = 2048
_NCHUNK = _S // _CHUNK

_iu_np, _ju_np = np.triu_indices(_N, k=1)
_MSEL = np.zeros((_EP, _N), np.float32)
_MSEL[np.arange(_E), _iu_np] = 1.0
_MSEL[np.arange(_E), _ju_np] = -1.0
_IU_ROW = np.zeros((1, _EP), np.float32)
_IU_ROW[0, :_E] = _iu_np
_JU_ROW = np.zeros((1, _EP), np.float32)
_JU_ROW[0, :_E] = _ju_np


def _body(feat_ref, noise_ref, msel_ref, iu_ref, ju_ref, out_ref, acc_ref):
    d = pl.program_id(0)
    s = pl.program_id(1)
    part = jnp.sum(feat_ref[0], axis=0, keepdims=True)

    @pl.when(s == 0)
    def _():
        acc_ref[pl.ds(d, 1), :] = part

    @pl.when(s != 0)
    def _():
        acc_ref[pl.ds(d, 1), :] += part

    @pl.when((d == _N - 1) & (s == _NCHUNK - 1))
    def _():
        means = acc_ref[...] * (1.0 / _S)
        delta = jnp.dot(msel_ref[...], means,
                        preferred_element_type=jnp.float32,
                        precision=lax.Precision.HIGHEST)
        w = jnp.sum(delta * delta, axis=1, keepdims=True)
        r0 = lax.broadcasted_iota(jnp.int32, (_EP, _EP), 0)
        c0 = lax.broadcasted_iota(jnp.int32, (_EP, _EP), 1)
        eye = (r0 == c0).astype(jnp.float32)
        w_row = lax.dot_general(w, eye, (((0,), (0,)), ((), ())),
                                precision=lax.Precision.HIGHEST)

        mean_w = jnp.sum(w_row) * (1.0 / _E)
        pert_row = w_row + noise_ref[...] * (mean_w * _RF)
        valid_row = c0[0:1, :] < _E
        neg = jnp.float32(-3e38)
        pert_row = jnp.where(valid_row, pert_row, neg)
        pert_col = lax.dot_general(eye, pert_row,
                                   (((1,), (1,)), ((), ())),
                                   precision=lax.Precision.HIGHEST)

        vj = jnp.broadcast_to(pert_row, (_EP, _EP))
        ve = jnp.broadcast_to(pert_col, (_EP, _EP))
        cmp = (vj > ve) | ((vj == ve) & (c0 < r0))
        rank = jnp.sum(cmp.astype(jnp.float32), axis=1, keepdims=True)

        onehot = ((rank == c0.astype(jnp.float32))
                  & (r0 < _E)).astype(jnp.float32)
        iu_out = lax.dot_general(iu_ref[...], onehot,
                                 (((1,), (0,)), ((), ())),
                                 precision=lax.Precision.HIGHEST)
        ju_out = lax.dot_general(ju_ref[...], onehot,
                                 (((1,), (0,)), ((), ())),
                                 precision=lax.Precision.HIGHEST)
        out_ref[0:1, :] = iu_out.astype(jnp.int32)
        out_ref[1:2, :] = ju_out.astype(jnp.int32)


def kernel(feature, noise):
    noise_row = jnp.zeros((1, _EP), jnp.float32).at[0, :_E].set(noise)
    out = pl.pallas_call(
        _body,
        grid=(_N, _NCHUNK),
        in_specs=[
            pl.BlockSpec((1, _CHUNK, _D), lambda d, s: (d, s, 0)),
            pl.BlockSpec((1, _EP), lambda d, s: (0, 0)),
            pl.BlockSpec((_EP, _N), lambda d, s: (0, 0)),
            pl.BlockSpec((1, _EP), lambda d, s: (0, 0)),
            pl.BlockSpec((1, _EP), lambda d, s: (0, 0)),
        ],
        out_specs=pl.BlockSpec((8, _EP), lambda d, s: (0, 0)),
        out_shape=jax.ShapeDtypeStruct((8, _EP), jnp.int32),
        scratch_shapes=[pltpu.VMEM((_N, _D), jnp.float32)],
    )(feature, noise_row, jnp.asarray(_MSEL), jnp.asarray(_IU_ROW),
      jnp.asarray(_JU_ROW))
    return out[:2, :_K]


# speedup vs baseline: 1.2749x; 1.2749x over previous
import numpy as np
import jax
import jax.numpy as jnp
from jax import lax
from jax.experimental import pallas as pl
from jax.experimental.pallas import tpu as pltpu

_N = 16
_S = 4096
_D = 256
_E = _N * (_N - 1) // 2
_EP = 128
_K = max(int(max(0.5 * 0.999, 0.4) * _E), 1)
_RF = 0.8
_CHUNK = 4096
_NCHUNK = _S // _CHUNK

_iu_np, _ju_np = np.triu_indices(_N, k=1)
_MSEL = np.zeros((_EP, _N), np.float32)
_MSEL[np.arange(_E), _iu_np] = 1.0
_MSEL[np.arange(_E), _ju_np] = -1.0
_IU_ROW = np.zeros((1, _EP), np.float32)
_IU_ROW[0, :_E] = _iu_np
_JU_ROW = np.zeros((1, _EP), np.float32)
_JU_ROW[0, :_E] = _ju_np


def _body(feat_ref, noise_ref, msel_ref, iu_ref, ju_ref, out_ref, acc_ref):
    d = pl.program_id(0)
    s = pl.program_id(1)
    part = jnp.sum(feat_ref[0], axis=0, keepdims=True)

    @pl.when(s == 0)
    def _():
        acc_ref[pl.ds(d, 1), :] = part

    @pl.when(s != 0)
    def _():
        acc_ref[pl.ds(d, 1), :] += part

    @pl.when((d == _N - 1) & (s == _NCHUNK - 1))
    def _():
        means = acc_ref[...] * (1.0 / _S)
        delta = jnp.dot(msel_ref[...], means,
                        preferred_element_type=jnp.float32,
                        precision=lax.Precision.HIGHEST)
        w = jnp.sum(delta * delta, axis=1, keepdims=True)
        r0 = lax.broadcasted_iota(jnp.int32, (_EP, _EP), 0)
        c0 = lax.broadcasted_iota(jnp.int32, (_EP, _EP), 1)
        eye = (r0 == c0).astype(jnp.float32)
        w_row = lax.dot_general(w, eye, (((0,), (0,)), ((), ())),
                                precision=lax.Precision.HIGHEST)

        mean_w = jnp.sum(w_row) * (1.0 / _E)
        pert_row = w_row + noise_ref[...] * (mean_w * _RF)
        valid_row = c0[0:1, :] < _E
        neg = jnp.float32(-3e38)
        pert_row = jnp.where(valid_row, pert_row, neg)
        pert_col = lax.dot_general(eye, pert_row,
                                   (((1,), (1,)), ((), ())),
                                   precision=lax.Precision.HIGHEST)

        vj = jnp.broadcast_to(pert_row, (_EP, _EP))
        ve = jnp.broadcast_to(pert_col, (_EP, _EP))
        cmp = (vj > ve) | ((vj == ve) & (c0 < r0))
        rank = jnp.sum(cmp.astype(jnp.float32), axis=1, keepdims=True)

        onehot = ((rank == c0.astype(jnp.float32))
                  & (r0 < _E)).astype(jnp.float32)
        iu_out = lax.dot_general(iu_ref[...], onehot,
                                 (((1,), (0,)), ((), ())),
                                 precision=lax.Precision.HIGHEST)
        ju_out = lax.dot_general(ju_ref[...], onehot,
                                 (((1,), (0,)), ((), ())),
                                 precision=lax.Precision.HIGHEST)
        out_ref[0:1, :] = iu_out.astype(jnp.int32)
        out_ref[1:2, :] = ju_out.astype(jnp.int32)


def kernel(feature, noise):
    noise_row = jnp.zeros((1, _EP), jnp.float32).at[0, :_E].set(noise)
    out = pl.pallas_call(
        _body,
        grid=(_N, _NCHUNK),
        in_specs=[
            pl.BlockSpec((1, _CHUNK, _D), lambda d, s: (d, s, 0)),
            pl.BlockSpec((1, _EP), lambda d, s: (0, 0)),
            pl.BlockSpec((_EP, _N), lambda d, s: (0, 0)),
            pl.BlockSpec((1, _EP), lambda d, s: (0, 0)),
            pl.BlockSpec((1, _EP), lambda d, s: (0, 0)),
        ],
        out_specs=pl.BlockSpec((8, _EP), lambda d, s: (0, 0)),
        out_shape=jax.ShapeDtypeStruct((8, _EP), jnp.int32),
        scratch_shapes=[pltpu.VMEM((_N, _D), jnp.float32)],
    )(feature, noise_row, jnp.asarray(_MSEL), jnp.asarray(_IU_ROW),
      jnp.asarray(_JU_ROW))
    return out[:2, :_K]


# speedup vs baseline: 1.4529x; 1.1396x over previous
import numpy as np
import jax
import jax.numpy as jnp
from jax import lax
from jax.experimental import pallas as pl
from jax.experimental.pallas import tpu as pltpu

_N = 16
_S = 4096
_D = 256
_E = _N * (_N - 1) // 2
_EP = 128
_K = max(int(max(0.5 * 0.999, 0.4) * _E), 1)
_RF = 0.8
_CHUNK = 4096
_NCHUNK = _S // _CHUNK
_DB = 2
_NDB = _N // _DB

_iu_np, _ju_np = np.triu_indices(_N, k=1)
_MSEL = np.zeros((_EP, _N), np.float32)
_MSEL[np.arange(_E), _iu_np] = 1.0
_MSEL[np.arange(_E), _ju_np] = -1.0
_IU_ROW = np.zeros((1, _EP), np.float32)
_IU_ROW[0, :_E] = _iu_np
_JU_ROW = np.zeros((1, _EP), np.float32)
_JU_ROW[0, :_E] = _ju_np


def _body(feat_ref, noise_ref, msel_ref, iu_ref, ju_ref, out_ref, acc_ref):
    d = pl.program_id(0)
    s = pl.program_id(1)
    for i in range(_DB):
        part = jnp.sum(feat_ref[i], axis=0, keepdims=True)

        @pl.when(s == 0)
        def _(part=part, i=i):
            acc_ref[pl.ds(d * _DB + i, 1), :] = part

        @pl.when(s != 0)
        def _(part=part, i=i):
            acc_ref[pl.ds(d * _DB + i, 1), :] += part

    @pl.when((d == _NDB - 1) & (s == _NCHUNK - 1))
    def _():
        means = acc_ref[...] * (1.0 / _S)
        delta = jnp.dot(msel_ref[...], means,
                        preferred_element_type=jnp.float32,
                        precision=lax.Precision.HIGHEST)
        w = jnp.sum(delta * delta, axis=1, keepdims=True)
        r0 = lax.broadcasted_iota(jnp.int32, (_EP, _EP), 0)
        c0 = lax.broadcasted_iota(jnp.int32, (_EP, _EP), 1)
        eye = (r0 == c0).astype(jnp.float32)
        w_row = lax.dot_general(w, eye, (((0,), (0,)), ((), ())),
                                precision=lax.Precision.HIGHEST)

        mean_w = jnp.sum(w_row) * (1.0 / _E)
        pert_row = w_row + noise_ref[...] * (mean_w * _RF)
        valid_row = c0[0:1, :] < _E
        neg = jnp.float32(-3e38)
        pert_row = jnp.where(valid_row, pert_row, neg)
        pert_col = lax.dot_general(eye, pert_row,
                                   (((1,), (1,)), ((), ())),
                                   precision=lax.Precision.HIGHEST)

        vj = jnp.broadcast_to(pert_row, (_EP, _EP))
        ve = jnp.broadcast_to(pert_col, (_EP, _EP))
        cmp = (vj > ve) | ((vj == ve) & (c0 < r0))
        rank = jnp.sum(cmp.astype(jnp.float32), axis=1, keepdims=True)

        onehot = ((rank == c0.astype(jnp.float32))
                  & (r0 < _E)).astype(jnp.float32)
        iu_out = lax.dot_general(iu_ref[...], onehot,
                                 (((1,), (0,)), ((), ())),
                                 precision=lax.Precision.HIGHEST)
        ju_out = lax.dot_general(ju_ref[...], onehot,
                                 (((1,), (0,)), ((), ())),
                                 precision=lax.Precision.HIGHEST)
        out_ref[0:1, :] = iu_out.astype(jnp.int32)
        out_ref[1:2, :] = ju_out.astype(jnp.int32)


def kernel(feature, noise):
    noise_row = jnp.zeros((1, _EP), jnp.float32).at[0, :_E].set(noise)
    out = pl.pallas_call(
        _body,
        grid=(_NDB, _NCHUNK),
        in_specs=[
            pl.BlockSpec((_DB, _CHUNK, _D), lambda d, s: (d, s, 0)),
            pl.BlockSpec((1, _EP), lambda d, s: (0, 0)),
            pl.BlockSpec((_EP, _N), lambda d, s: (0, 0)),
            pl.BlockSpec((1, _EP), lambda d, s: (0, 0)),
            pl.BlockSpec((1, _EP), lambda d, s: (0, 0)),
        ],
        out_specs=pl.BlockSpec((8, _EP), lambda d, s: (0, 0)),
        out_shape=jax.ShapeDtypeStruct((8, _EP), jnp.int32),
        scratch_shapes=[pltpu.VMEM((_N, _D), jnp.float32)],
    )(feature, noise_row, jnp.asarray(_MSEL), jnp.asarray(_IU_ROW),
      jnp.asarray(_JU_ROW))
    return out[:2, :_K]
